# baseline (speedup 1.0000x reference)
import functools

import jax
import jax.numpy as jnp
from jax import lax
from jax.experimental import pallas as pl
from jax.experimental.pallas import tpu as pltpu
from jax.experimental.pallas import tpu_sc as plsc


def _make_sc_gather(n_rows: int, emb: int, idx_dtype, num_cores: int,
                    num_subcores: int):
    nw = num_cores * num_subcores
    rows_per_w = max(8, n_rows // nw)
    n_active = n_rows // rows_per_w
    mesh = plsc.VectorSubcoreMesh(core_axis_name="c", subcore_axis_name="s")

    @functools.partial(
        pl.kernel,
        mesh=mesh,
        out_type=jax.ShapeDtypeStruct((n_rows, emb), jnp.float32),
        scratch_types=[
            pltpu.VMEM((rows_per_w,), idx_dtype),
            pltpu.VMEM((rows_per_w, emb), jnp.float32),
            pltpu.SemaphoreType.DMA,
        ],
        compiler_params=pltpu.CompilerParams(use_tc_tiling_on_sc=False),
    )
    def gather(idx_hbm, table_hbm, pe_hbm, idx_v, rows_v, sem):
        wid = lax.axis_index("s") * num_cores + lax.axis_index("c")

        @pl.when(wid < n_active)
        def _():
            base = wid * rows_per_w
            pltpu.sync_copy(idx_hbm.at[pl.ds(base, rows_per_w)], idx_v)
            pltpu.async_copy(table_hbm.at[idx_v], rows_v, sem).wait()
            pltpu.sync_copy(rows_v, pe_hbm.at[pl.ds(base, rows_per_w)])

    return gather


def _add_body(x_ref, pe_ref, o_ref):
    o_ref[...] = x_ref[...] + pe_ref[...][None, :, None, :]


def kernel(x, input_channels, pos_embed):
    B, N, P, E = x.shape
    input_channels = input_channels.astype(jnp.int32)

    info = plsc.get_sparse_core_info()
    gather = _make_sc_gather(N, E, jnp.int32, info.num_cores,
                             info.num_subcores)
    pe = gather(input_channels, pos_embed)

    NCHUNK = 16
    out = pl.pallas_call(
        _add_body,
        grid=(B, N // NCHUNK),
        in_specs=[
            pl.BlockSpec((1, NCHUNK, P, E), lambda b, j: (b, j, 0, 0)),
            pl.BlockSpec((NCHUNK, E), lambda b, j: (j, 0)),
        ],
        out_specs=pl.BlockSpec((1, NCHUNK, P, E), lambda b, j: (b, j, 0, 0)),
        out_shape=jax.ShapeDtypeStruct((B, N, P, E), jnp.float32),
    )(x, pe)
    return out
